# baseline (speedup 1.0000x reference)
import numpy as np
import jax
import jax.numpy as jnp
from jax import lax
from jax.experimental import pallas as pl
from jax.experimental.pallas import tpu as pltpu
from jax.experimental.pallas import tpu_sc as plsc

N = 100000
D = 128
B = 512
MAX_DEG = 32
S1 = 10
S2 = 25
NB1 = B * S1
NC, NSC = 2, 16
NW = NC * NSC
PB = B // NW
P1 = NB1 // NW
APR = D // MAX_DEG
NCH = 40
GPC = 4

def _mesh():
  return plsc.VectorSubcoreMesh(core_axis_name="c", subcore_axis_name="s",
                                num_cores=NC, num_subcores=NSC)


def _sc1_body(batch_hbm, adj4_hbm, feat_hbm, h0_out, adj1_out,
              batch_v, bd4_v, h0b_v, adj1_v, sem):
  wid = lax.axis_index("s") * NC + lax.axis_index("c")
  pltpu.sync_copy(batch_hbm.at[pl.ds(wid * PB, PB)], batch_v)
  cp0 = pltpu.async_copy(feat_hbm.at[batch_v], h0b_v, sem)
  bd4_v[...] = lax.shift_right_logical(batch_v[...], 2)
  cp1 = pltpu.async_copy(adj4_hbm.at[bd4_v], adj1_v, sem)
  cp0.wait()
  cp1.wait()
  pltpu.sync_copy(h0b_v, h0_out.at[pl.ds(wid * PB, PB)])
  pltpu.sync_copy(adj1_v, adj1_out.at[pl.ds(wid * PB, PB)])


def _sc1(batch, adj4, features):
  f = pl.kernel(
      _sc1_body,
      out_type=(jax.ShapeDtypeStruct((B, D), jnp.float32),
                jax.ShapeDtypeStruct((B, D), jnp.int32)),
      mesh=_mesh(),
      scratch_types=[
          pltpu.VMEM((PB,), jnp.int32),
          pltpu.VMEM((PB,), jnp.int32),
          pltpu.VMEM((PB, D), jnp.float32),
          pltpu.VMEM((PB, D), jnp.int32),
          pltpu.SemaphoreType.DMA,
      ])
  return f(batch, adj4, features)


def _sc2_body(s1_hbm, s1d4_hbm, adj4_hbm, feat_hbm, h1_out, adj2_out,
              s1_v, s1d4_v, h1b_v, adj2_v, sem):
  wid = lax.axis_index("s") * NC + lax.axis_index("c")
  pltpu.sync_copy(s1_hbm.at[pl.ds(wid * P1, P1)], s1_v)
  pltpu.sync_copy(s1d4_hbm.at[pl.ds(wid * P1, P1)], s1d4_v)
  cps = []
  for half in range(2):
    sl = pl.ds(half * 80, 80)
    cps.append(pltpu.async_copy(feat_hbm.at[s1_v.at[sl]],
                                h1b_v.at[sl], sem))
    cps.append(pltpu.async_copy(adj4_hbm.at[s1d4_v.at[sl]],
                                adj2_v.at[sl], sem))
  for cp in cps:
    cp.wait()
  pltpu.sync_copy(h1b_v, h1_out.at[pl.ds(wid * P1, P1)])
  pltpu.sync_copy(adj2_v, adj2_out.at[pl.ds(wid * P1, P1)])


def _sc2(s1, s1d4, adj4, features):
  f = pl.kernel(
      _sc2_body,
      out_type=(jax.ShapeDtypeStruct((NB1, D), jnp.float32),
                jax.ShapeDtypeStruct((NB1, D), jnp.int32)),
      mesh=_mesh(),
      scratch_types=[
          pltpu.VMEM((P1,), jnp.int32),
          pltpu.VMEM((P1,), jnp.int32),
          pltpu.VMEM((P1, D), jnp.float32),
          pltpu.VMEM((P1, D), jnp.int32),
          pltpu.SemaphoreType.DMA,
      ])
  return f(s1, s1d4, adj4, features)


def _sc3_body(idx2_hbm, feat_hbm, nsum_out, idx2_v, gbuf_v, sums_v, sem):
  wid = lax.axis_index("s") * NC + lax.axis_index("c")
  pltpu.sync_copy(idx2_hbm.at[wid], idx2_v)

  def chunk_body(c, _):
    pltpu.async_copy(feat_hbm.at[idx2_v.at[c]], gbuf_v, sem).wait()
    for g in range(GPC):
      for h in range(D // 16):
        sl = pl.ds(h * 16, 16)
        acc = gbuf_v[g * S2, sl]
        for r in range(1, S2):
          acc = acc + gbuf_v[g * S2 + r, sl]
        sums_v[c * GPC + g, sl] = acc
    return 0
  lax.fori_loop(0, NCH, chunk_body, 0)
  pltpu.sync_copy(sums_v, nsum_out.at[pl.ds(wid * P1, P1)])


def _sc3(idx2, features):
  f = pl.kernel(
      _sc3_body,
      out_type=(jax.ShapeDtypeStruct((NB1, D), jnp.float32),),
      mesh=_mesh(),
      scratch_types=[
          pltpu.VMEM((NCH, D), jnp.int32),
          pltpu.VMEM((D, D), jnp.float32),
          pltpu.VMEM((P1, D), jnp.float32),
          pltpu.SemaphoreType.DMA,
      ])
  return f(idx2, features)[0]


def _extract(adjrows_f, nodes, nsamp):
  picked = jnp.zeros((adjrows_f.shape[0], nsamp), jnp.float32)
  col = jax.lax.broadcasted_iota(jnp.int32, (D, nsamp), 0)
  samp = jax.lax.broadcasted_iota(jnp.int32, (D, nsamp), 1)
  nmod = nodes - 4.0 * jnp.floor(nodes * 0.25)
  for r in range(APR):
    oh = jnp.where(col == r * MAX_DEG + samp, 1.0, 0.0)
    sel = jnp.dot(adjrows_f, oh, preferred_element_type=jnp.float32,
                  precision=lax.Precision.HIGHEST)
    picked = picked + jnp.where(nmod == float(r), 1.0, 0.0) * sel
  return picked


def _tca_body(adj1f_ref, bf_ref, s1_ref, s1d4_ref):
  picked = _extract(adj1f_ref[...], bf_ref[...], S1)
  s1_ref[...] = picked.astype(jnp.int32)
  s1d4_ref[...] = jnp.floor(picked * 0.25).astype(jnp.int32)


def _tcb_body(adj2f_ref, s1f_ref, s2_ref):
  picked = _extract(adj2f_ref[...], s1f_ref[...], 32)
  s2_ref[...] = picked.astype(jnp.int32)


def _tc1_body(h1f_ref, nsum_ref, ws_ref, wn_ref, out_ref):
  a = jnp.dot(h1f_ref[...], ws_ref[...], preferred_element_type=jnp.float32)
  nm = nsum_ref[...] * (1.0 / S2)
  b = jnp.dot(nm, wn_ref[...], preferred_element_type=jnp.float32)
  out_ref[...] = jnp.maximum(jnp.concatenate([a, b], axis=1), 0.0)


def _tc2_body(h0f_ref, h1f3_ref, h13_ref, ws0_ref, wn0_ref,
              ws1_ref, wn1_ref, wp_ref, bp_ref, out_ref):
  nm1 = jnp.mean(h1f3_ref[...], axis=1)
  a = jnp.dot(h0f_ref[...], ws0_ref[...], preferred_element_type=jnp.float32)
  b = jnp.dot(nm1, wn0_ref[...], preferred_element_type=jnp.float32)
  h0 = jnp.maximum(jnp.concatenate([a, b], axis=1), 0.0)
  h1m = jnp.mean(h13_ref[...], axis=1)
  out = jnp.concatenate(
      [jnp.dot(h0, ws1_ref[...], preferred_element_type=jnp.float32),
       jnp.dot(h1m, wn1_ref[...], preferred_element_type=jnp.float32)],
      axis=1)
  nrm = jnp.sqrt(jnp.sum(out * out, axis=1, keepdims=True))
  out = out / jnp.maximum(nrm, 1e-12)
  logits = jnp.dot(out, wp_ref[...], preferred_element_type=jnp.float32)
  logits = logits + bp_ref[...]
  m = jnp.max(logits, axis=1, keepdims=True)
  e = jnp.exp(logits - m)
  out_ref[...] = e / jnp.sum(e, axis=1, keepdims=True)


def kernel(batch, features, adj, W_self_0, W_neigh_0, W_self_1, W_neigh_1,
           W_pred, b_pred):
  adj4 = adj.reshape(N // APR, D)

  h0f, adj1_rows = _sc1(batch, adj4, features)

  s1, s1d4 = pl.pallas_call(
      _tca_body,
      out_shape=(jax.ShapeDtypeStruct((B, S1), jnp.int32),
                 jax.ShapeDtypeStruct((B, S1), jnp.int32)),
  )(adj1_rows.astype(jnp.float32), batch.astype(jnp.float32).reshape(B, 1))

  h1f, adj2_rows = _sc2(s1.reshape(NB1), s1d4.reshape(NB1), adj4, features)

  s2p = pl.pallas_call(
      _tcb_body,
      out_shape=jax.ShapeDtypeStruct((NB1, 32), jnp.int32),
  )(adj2_rows.astype(jnp.float32), s1.astype(jnp.float32).reshape(NB1, 1))

  s2 = s2p[:, :S2].reshape(NW, NCH, 100)
  idx2 = jnp.concatenate(
      [s2, jnp.broadcast_to(s2[:, :, 99:100], (NW, NCH, 28))], axis=2)

  nsum2 = _sc3(idx2, features)

  h1 = pl.pallas_call(
      _tc1_body,
      out_shape=jax.ShapeDtypeStruct((NB1, 2 * D), jnp.float32),
  )(h1f, nsum2, W_self_0, W_neigh_0)

  preds = pl.pallas_call(
      _tc2_body,
      out_shape=jax.ShapeDtypeStruct((B, 50), jnp.float32),
  )(h0f, h1f.reshape(B, S1, D), h1.reshape(B, S1, 2 * D),
    W_self_0, W_neigh_0, W_self_1, W_neigh_1, W_pred,
    b_pred.reshape(1, 50))
  return preds


# speedup vs baseline: 1.0606x; 1.0606x over previous
import numpy as np
import jax
import jax.numpy as jnp
from jax import lax
from jax.experimental import pallas as pl
from jax.experimental.pallas import tpu as pltpu
from jax.experimental.pallas import tpu_sc as plsc

N = 100000
D = 128
B = 512
MAX_DEG = 32
S1 = 10
S2 = 25
NB1 = B * S1
NC, NSC = 2, 16
NW = NC * NSC
PB = B // NW
P1 = NB1 // NW
APR = D // MAX_DEG
NCH = 40
GPC = 4

def _mesh():
  return plsc.VectorSubcoreMesh(core_axis_name="c", subcore_axis_name="s",
                                num_cores=NC, num_subcores=NSC)


def _sc1_body(batch_hbm, adj4_hbm, feat_hbm, h0_out, adj1_out,
              batch_v, bd4_v, h0b_v, adj1_v, sem):
  wid = lax.axis_index("s") * NC + lax.axis_index("c")
  pltpu.sync_copy(batch_hbm.at[pl.ds(wid * PB, PB)], batch_v)
  cp0 = pltpu.async_copy(feat_hbm.at[batch_v], h0b_v, sem)
  bd4_v[...] = lax.shift_right_logical(batch_v[...], 2)
  cp1 = pltpu.async_copy(adj4_hbm.at[bd4_v], adj1_v, sem)
  cp0.wait()
  cp1.wait()
  pltpu.sync_copy(h0b_v, h0_out.at[pl.ds(wid * PB, PB)])
  pltpu.sync_copy(adj1_v, adj1_out.at[pl.ds(wid * PB, PB)])


def _sc1(batch, adj4, features):
  f = pl.kernel(
      _sc1_body,
      out_type=(jax.ShapeDtypeStruct((B, D), jnp.float32),
                jax.ShapeDtypeStruct((B, D), jnp.int32)),
      mesh=_mesh(),
      scratch_types=[
          pltpu.VMEM((PB,), jnp.int32),
          pltpu.VMEM((PB,), jnp.int32),
          pltpu.VMEM((PB, D), jnp.float32),
          pltpu.VMEM((PB, D), jnp.int32),
          pltpu.SemaphoreType.DMA,
      ])
  return f(batch, adj4, features)


def _sc2_body(s1_hbm, s1d4_hbm, adj4_hbm, feat_hbm, h1_out, adj2_out,
              s1_v, s1d4_v, h1b_v, adj2_v, sem):
  wid = lax.axis_index("s") * NC + lax.axis_index("c")
  pltpu.sync_copy(s1_hbm.at[pl.ds(wid * P1, P1)], s1_v)
  pltpu.sync_copy(s1d4_hbm.at[pl.ds(wid * P1, P1)], s1d4_v)
  cps = []
  for half in range(2):
    sl = pl.ds(half * 80, 80)
    cps.append(pltpu.async_copy(feat_hbm.at[s1_v.at[sl]],
                                h1b_v.at[sl], sem))
    cps.append(pltpu.async_copy(adj4_hbm.at[s1d4_v.at[sl]],
                                adj2_v.at[sl], sem))
  for cp in cps:
    cp.wait()
  pltpu.sync_copy(h1b_v, h1_out.at[pl.ds(wid * P1, P1)])
  pltpu.sync_copy(adj2_v, adj2_out.at[pl.ds(wid * P1, P1)])


def _sc2(s1, s1d4, adj4, features):
  f = pl.kernel(
      _sc2_body,
      out_type=(jax.ShapeDtypeStruct((NB1, D), jnp.float32),
                jax.ShapeDtypeStruct((NB1, D), jnp.int32)),
      mesh=_mesh(),
      scratch_types=[
          pltpu.VMEM((P1,), jnp.int32),
          pltpu.VMEM((P1,), jnp.int32),
          pltpu.VMEM((P1, D), jnp.float32),
          pltpu.VMEM((P1, D), jnp.int32),
          pltpu.SemaphoreType.DMA,
      ])
  return f(s1, s1d4, adj4, features)


def _sc3_body(idx2_hbm, feat_hbm, nsum_out, idx2_v, gbuf_v, gbuf2_v, sums_v, sem):
  wid = lax.axis_index("s") * NC + lax.axis_index("c")
  pltpu.sync_copy(idx2_hbm.at[wid], idx2_v)

  def chunk_body(c, _):
    cb = c * 2
    cp0 = pltpu.async_copy(feat_hbm.at[idx2_v.at[cb]], gbuf_v, sem)
    cp1 = pltpu.async_copy(feat_hbm.at[idx2_v.at[cb + 1]], gbuf2_v, sem)
    for half, (cp, buf) in enumerate(((cp0, gbuf_v), (cp1, gbuf2_v))):
      cp.wait()
      for g in range(GPC):
        for h in range(D // 16):
          sl = pl.ds(h * 16, 16)
          acc = buf[g * S2, sl]
          for r in range(1, S2):
            acc = acc + buf[g * S2 + r, sl]
          sums_v[(cb + half) * GPC + g, sl] = acc
    return 0
  lax.fori_loop(0, NCH // 2, chunk_body, 0)
  pltpu.sync_copy(sums_v, nsum_out.at[pl.ds(wid * P1, P1)])


def _sc3(idx2, features):
  f = pl.kernel(
      _sc3_body,
      out_type=(jax.ShapeDtypeStruct((NB1, D), jnp.float32),),
      mesh=_mesh(),
      scratch_types=[
          pltpu.VMEM((NCH, D), jnp.int32),
          pltpu.VMEM((D, D), jnp.float32),
          pltpu.VMEM((D, D), jnp.float32),
          pltpu.VMEM((P1, D), jnp.float32),
          pltpu.SemaphoreType.DMA,
      ])
  return f(idx2, features)[0]


def _extract(adjrows_f, nodes, nsamp):
  picked = jnp.zeros((adjrows_f.shape[0], nsamp), jnp.float32)
  col = jax.lax.broadcasted_iota(jnp.int32, (D, nsamp), 0)
  samp = jax.lax.broadcasted_iota(jnp.int32, (D, nsamp), 1)
  nmod = nodes - 4.0 * jnp.floor(nodes * 0.25)
  for r in range(APR):
    oh = jnp.where(col == r * MAX_DEG + samp, 1.0, 0.0)
    sel = jnp.dot(adjrows_f, oh, preferred_element_type=jnp.float32,
                  precision=lax.Precision.HIGHEST)
    picked = picked + jnp.where(nmod == float(r), 1.0, 0.0) * sel
  return picked


def _tca_body(adj1f_ref, bf_ref, s1_ref, s1d4_ref):
  picked = _extract(adj1f_ref[...], bf_ref[...], S1)
  s1_ref[...] = picked.astype(jnp.int32)
  s1d4_ref[...] = jnp.floor(picked * 0.25).astype(jnp.int32)


def _tcb_body(adj2f_ref, s1f_ref, s2_ref):
  picked = _extract(adj2f_ref[...], s1f_ref[...], 32)
  s2_ref[...] = picked.astype(jnp.int32)


def _tc1_body(h1f_ref, nsum_ref, ws_ref, wn_ref, out_ref):
  a = jnp.dot(h1f_ref[...], ws_ref[...], preferred_element_type=jnp.float32)
  nm = nsum_ref[...] * (1.0 / S2)
  b = jnp.dot(nm, wn_ref[...], preferred_element_type=jnp.float32)
  out_ref[...] = jnp.maximum(jnp.concatenate([a, b], axis=1), 0.0)


def _tc2_body(h0f_ref, h1f3_ref, h13_ref, ws0_ref, wn0_ref,
              ws1_ref, wn1_ref, wp_ref, bp_ref, out_ref):
  nm1 = jnp.mean(h1f3_ref[...], axis=1)
  a = jnp.dot(h0f_ref[...], ws0_ref[...], preferred_element_type=jnp.float32)
  b = jnp.dot(nm1, wn0_ref[...], preferred_element_type=jnp.float32)
  h0 = jnp.maximum(jnp.concatenate([a, b], axis=1), 0.0)
  h1m = jnp.mean(h13_ref[...], axis=1)
  out = jnp.concatenate(
      [jnp.dot(h0, ws1_ref[...], preferred_element_type=jnp.float32),
       jnp.dot(h1m, wn1_ref[...], preferred_element_type=jnp.float32)],
      axis=1)
  nrm = jnp.sqrt(jnp.sum(out * out, axis=1, keepdims=True))
  out = out / jnp.maximum(nrm, 1e-12)
  logits = jnp.dot(out, wp_ref[...], preferred_element_type=jnp.float32)
  logits = logits + bp_ref[...]
  m = jnp.max(logits, axis=1, keepdims=True)
  e = jnp.exp(logits - m)
  out_ref[...] = e / jnp.sum(e, axis=1, keepdims=True)


def kernel(batch, features, adj, W_self_0, W_neigh_0, W_self_1, W_neigh_1,
           W_pred, b_pred):
  adj4 = adj.reshape(N // APR, D)

  h0f, adj1_rows = _sc1(batch, adj4, features)

  s1, s1d4 = pl.pallas_call(
      _tca_body,
      out_shape=(jax.ShapeDtypeStruct((B, S1), jnp.int32),
                 jax.ShapeDtypeStruct((B, S1), jnp.int32)),
  )(adj1_rows.astype(jnp.float32), batch.astype(jnp.float32).reshape(B, 1))

  h1f, adj2_rows = _sc2(s1.reshape(NB1), s1d4.reshape(NB1), adj4, features)

  s2p = pl.pallas_call(
      _tcb_body,
      out_shape=jax.ShapeDtypeStruct((NB1, 32), jnp.int32),
  )(adj2_rows.astype(jnp.float32), s1.astype(jnp.float32).reshape(NB1, 1))

  s2 = s2p[:, :S2].reshape(NW, NCH, 100)
  idx2 = jnp.concatenate(
      [s2, jnp.broadcast_to(s2[:, :, 99:100], (NW, NCH, 28))], axis=2)

  nsum2 = _sc3(idx2, features)

  h1 = pl.pallas_call(
      _tc1_body,
      out_shape=jax.ShapeDtypeStruct((NB1, 2 * D), jnp.float32),
  )(h1f, nsum2, W_self_0, W_neigh_0)

  preds = pl.pallas_call(
      _tc2_body,
      out_shape=jax.ShapeDtypeStruct((B, 50), jnp.float32),
  )(h0f, h1f.reshape(B, S1, D), h1.reshape(B, S1, 2 * D),
    W_self_0, W_neigh_0, W_self_1, W_neigh_1, W_pred,
    b_pred.reshape(1, 50))
  return preds


# speedup vs baseline: 1.1863x; 1.1185x over previous
import numpy as np
import jax
import jax.numpy as jnp
from jax import lax
from jax.experimental import pallas as pl
from jax.experimental.pallas import tpu as pltpu
from jax.experimental.pallas import tpu_sc as plsc

N = 100000
D = 128
B = 512
MAX_DEG = 32
S1 = 10
S2 = 25
NB1 = B * S1
NC, NSC = 2, 16
NW = NC * NSC
PB = B // NW
P1 = NB1 // NW
APR = D // MAX_DEG
NCH = 32
GPC = 5

def _mesh():
  return plsc.VectorSubcoreMesh(core_axis_name="c", subcore_axis_name="s",
                                num_cores=NC, num_subcores=NSC)


def _sc1_body(batch_hbm, adj4_hbm, feat_hbm, h0_out, adj1_out,
              batch_v, bd4_v, h0b_v, adj1_v, sem):
  wid = lax.axis_index("s") * NC + lax.axis_index("c")
  pltpu.sync_copy(batch_hbm.at[pl.ds(wid * PB, PB)], batch_v)
  cp0 = pltpu.async_copy(feat_hbm.at[batch_v], h0b_v, sem)
  bd4_v[...] = lax.shift_right_logical(batch_v[...], 2)
  cp1 = pltpu.async_copy(adj4_hbm.at[bd4_v], adj1_v, sem)
  cp0.wait()
  cp1.wait()
  pltpu.sync_copy(h0b_v, h0_out.at[pl.ds(wid * PB, PB)])
  pltpu.sync_copy(adj1_v, adj1_out.at[pl.ds(wid * PB, PB)])


def _sc1(batch, adj4, features):
  f = pl.kernel(
      _sc1_body,
      out_type=(jax.ShapeDtypeStruct((B, D), jnp.float32),
                jax.ShapeDtypeStruct((B, D), jnp.int32)),
      mesh=_mesh(),
      scratch_types=[
          pltpu.VMEM((PB,), jnp.int32),
          pltpu.VMEM((PB,), jnp.int32),
          pltpu.VMEM((PB, D), jnp.float32),
          pltpu.VMEM((PB, D), jnp.int32),
          pltpu.SemaphoreType.DMA,
      ])
  return f(batch, adj4, features)


def _sc2_body(s1_hbm, s1d4_hbm, adj4_hbm, feat_hbm, h1_out, adj2_out,
              s1_v, s1d4_v, h1b_v, adj2_v, sem):
  wid = lax.axis_index("s") * NC + lax.axis_index("c")
  pltpu.sync_copy(s1_hbm.at[pl.ds(wid * P1, P1)], s1_v)
  pltpu.sync_copy(s1d4_hbm.at[pl.ds(wid * P1, P1)], s1d4_v)
  cps = []
  for half in range(2):
    sl = pl.ds(half * 80, 80)
    cps.append(pltpu.async_copy(feat_hbm.at[s1_v.at[sl]],
                                h1b_v.at[sl], sem))
    cps.append(pltpu.async_copy(adj4_hbm.at[s1d4_v.at[sl]],
                                adj2_v.at[sl], sem))
  for cp in cps:
    cp.wait()
  pltpu.sync_copy(h1b_v, h1_out.at[pl.ds(wid * P1, P1)])
  pltpu.sync_copy(adj2_v, adj2_out.at[pl.ds(wid * P1, P1)])


def _sc2(s1, s1d4, adj4, features):
  f = pl.kernel(
      _sc2_body,
      out_type=(jax.ShapeDtypeStruct((NB1, D), jnp.float32),
                jax.ShapeDtypeStruct((NB1, D), jnp.int32)),
      mesh=_mesh(),
      scratch_types=[
          pltpu.VMEM((P1,), jnp.int32),
          pltpu.VMEM((P1,), jnp.int32),
          pltpu.VMEM((P1, D), jnp.float32),
          pltpu.VMEM((P1, D), jnp.int32),
          pltpu.SemaphoreType.DMA,
      ])
  return f(s1, s1d4, adj4, features)


def _sc3_body(idx2_hbm, feat_hbm, nsum_out, idx2_v, gbuf_v, gbuf2_v, sums_v, sem):
  wid = lax.axis_index("s") * NC + lax.axis_index("c")
  pltpu.sync_copy(idx2_hbm.at[wid], idx2_v)

  def chunk_body(c, _):
    cb = c * 2
    cp0 = pltpu.async_copy(feat_hbm.at[idx2_v.at[cb]], gbuf_v, sem)
    cp1 = pltpu.async_copy(feat_hbm.at[idx2_v.at[cb + 1]], gbuf2_v, sem)
    for half, (cp, buf) in enumerate(((cp0, gbuf_v), (cp1, gbuf2_v))):
      cp.wait()
      for g in range(GPC):
        for h in range(D // 16):
          sl = pl.ds(h * 16, 16)
          acc = buf[g * S2, sl]
          for r in range(1, S2):
            acc = acc + buf[g * S2 + r, sl]
          sums_v[(cb + half) * GPC + g, sl] = acc
    return 0
  lax.fori_loop(0, NCH // 2, chunk_body, 0)
  pltpu.sync_copy(sums_v, nsum_out.at[pl.ds(wid * P1, P1)])


def _sc3(idx2, features):
  f = pl.kernel(
      _sc3_body,
      out_type=(jax.ShapeDtypeStruct((NB1, D), jnp.float32),),
      mesh=_mesh(),
      scratch_types=[
          pltpu.VMEM((NCH, D), jnp.int32),
          pltpu.VMEM((D, D), jnp.float32),
          pltpu.VMEM((D, D), jnp.float32),
          pltpu.VMEM((P1, D), jnp.float32),
          pltpu.SemaphoreType.DMA,
      ])
  return f(idx2, features)[0]


def _extract(adjrows_f, nodes, nsamp):
  picked = jnp.zeros((adjrows_f.shape[0], nsamp), jnp.float32)
  col = jax.lax.broadcasted_iota(jnp.int32, (D, nsamp), 0)
  samp = jax.lax.broadcasted_iota(jnp.int32, (D, nsamp), 1)
  nmod = nodes - 4.0 * jnp.floor(nodes * 0.25)
  for r in range(APR):
    oh = jnp.where(col == r * MAX_DEG + samp, 1.0, 0.0)
    sel = jnp.dot(adjrows_f, oh, preferred_element_type=jnp.float32,
                  precision=lax.Precision.HIGHEST)
    picked = picked + jnp.where(nmod == float(r), 1.0, 0.0) * sel
  return picked


def _tca_body(adj1f_ref, bf_ref, s1_ref, s1d4_ref):
  picked = _extract(adj1f_ref[...], bf_ref[...], S1)
  s1_ref[...] = picked.astype(jnp.int32)
  s1d4_ref[...] = jnp.floor(picked * 0.25).astype(jnp.int32)


def _tcb_body(adj2f_ref, s1f_ref, s2_ref):
  picked = _extract(adj2f_ref[...], s1f_ref[...], 32)
  s2_ref[...] = picked.astype(jnp.int32)


def _tc1_body(h1f_ref, nsum_ref, ws_ref, wn_ref, out_ref):
  a = jnp.dot(h1f_ref[...], ws_ref[...], preferred_element_type=jnp.float32)
  nm = nsum_ref[...] * (1.0 / S2)
  b = jnp.dot(nm, wn_ref[...], preferred_element_type=jnp.float32)
  out_ref[...] = jnp.maximum(jnp.concatenate([a, b], axis=1), 0.0)


def _tc2_body(h0f_ref, h1f3_ref, h13_ref, ws0_ref, wn0_ref,
              ws1_ref, wn1_ref, wp_ref, bp_ref, out_ref):
  nm1 = jnp.mean(h1f3_ref[...], axis=1)
  a = jnp.dot(h0f_ref[...], ws0_ref[...], preferred_element_type=jnp.float32)
  b = jnp.dot(nm1, wn0_ref[...], preferred_element_type=jnp.float32)
  h0 = jnp.maximum(jnp.concatenate([a, b], axis=1), 0.0)
  h1m = jnp.mean(h13_ref[...], axis=1)
  out = jnp.concatenate(
      [jnp.dot(h0, ws1_ref[...], preferred_element_type=jnp.float32),
       jnp.dot(h1m, wn1_ref[...], preferred_element_type=jnp.float32)],
      axis=1)
  nrm = jnp.sqrt(jnp.sum(out * out, axis=1, keepdims=True))
  out = out / jnp.maximum(nrm, 1e-12)
  logits = jnp.dot(out, wp_ref[...], preferred_element_type=jnp.float32)
  logits = logits + bp_ref[...]
  m = jnp.max(logits, axis=1, keepdims=True)
  e = jnp.exp(logits - m)
  out_ref[...] = e / jnp.sum(e, axis=1, keepdims=True)


def kernel(batch, features, adj, W_self_0, W_neigh_0, W_self_1, W_neigh_1,
           W_pred, b_pred):
  adj4 = adj.reshape(N // APR, D)

  h0f, adj1_rows = _sc1(batch, adj4, features)

  s1, s1d4 = pl.pallas_call(
      _tca_body,
      out_shape=(jax.ShapeDtypeStruct((B, S1), jnp.int32),
                 jax.ShapeDtypeStruct((B, S1), jnp.int32)),
  )(adj1_rows.astype(jnp.float32), batch.astype(jnp.float32).reshape(B, 1))

  h1f, adj2_rows = _sc2(s1.reshape(NB1), s1d4.reshape(NB1), adj4, features)

  s2p = pl.pallas_call(
      _tcb_body,
      out_shape=jax.ShapeDtypeStruct((NB1, 32), jnp.int32),
  )(adj2_rows.astype(jnp.float32), s1.astype(jnp.float32).reshape(NB1, 1))

  s2 = s2p[:, :S2].reshape(NW, NCH, 125)
  idx2 = jnp.concatenate(
      [s2, jnp.broadcast_to(s2[:, :, 124:125], (NW, NCH, 3))], axis=2)

  nsum2 = _sc3(idx2, features)

  h1 = pl.pallas_call(
      _tc1_body,
      out_shape=jax.ShapeDtypeStruct((NB1, 2 * D), jnp.float32),
  )(h1f, nsum2, W_self_0, W_neigh_0)

  preds = pl.pallas_call(
      _tc2_body,
      out_shape=jax.ShapeDtypeStruct((B, 50), jnp.float32),
  )(h0f, h1f.reshape(B, S1, D), h1.reshape(B, S1, 2 * D),
    W_self_0, W_neigh_0, W_self_1, W_neigh_1, W_pred,
    b_pred.reshape(1, 50))
  return preds
